# baseline (speedup 1.0000x reference)
import functools

import jax
import jax.numpy as jnp
from jax import lax
from jax.experimental import pallas as pl
from jax.experimental.pallas import tpu as pltpu
from jax.experimental.pallas import tpu_sc as plsc

_IBIG = 2**30


def _local_topk(x, iota, k, axis, largest):
    vals, idxs = [], []
    for j in range(k):
        if largest:
            m = jnp.max(x, axis=axis, keepdims=True)
        else:
            m = jnp.min(x, axis=axis, keepdims=True)
        i = jnp.min(jnp.where(x == m, iota, _IBIG), axis=axis, keepdims=True)
        vals.append(m)
        idxs.append(i)
        if j < k - 1:
            fill = -jnp.inf if largest else jnp.inf
            x = jnp.where(iota == i, fill, x)
    return jnp.concatenate(vals, axis=axis), jnp.concatenate(idxs, axis=axis)


def _lex_merge(vals, idxs, k, axis, largest):
    out_v, out_i = [], []
    for j in range(k):
        if largest:
            m = jnp.max(vals, axis=axis, keepdims=True)
        else:
            m = jnp.min(vals, axis=axis, keepdims=True)
        i = jnp.min(jnp.where(vals == m, idxs, _IBIG), axis=axis, keepdims=True)
        out_v.append(m)
        out_i.append(i)
        if j < k - 1:
            fill = -jnp.inf if largest else jnp.inf
            vals = jnp.where((vals == m) & (idxs == i), fill, vals)
    return jnp.concatenate(out_v, axis=axis), jnp.concatenate(out_i, axis=axis)


def _topk0_body(x_ref, vals_ref, idx_ref):
    C = 8192
    rv = ri = None
    for c in range(32768 // C):
        x = x_ref[:, pl.ds(c * C, C)]
        iota = jax.lax.broadcasted_iota(jnp.int32, x.shape, 1) + c * C
        lv, li = _local_topk(x, iota, 2, 1, True)
        if rv is None:
            rv, ri = lv, li
        else:
            rv, ri = _lex_merge(
                jnp.concatenate([rv, lv], axis=1),
                jnp.concatenate([ri, li], axis=1),
                2, 1, True)
    vals_ref[...] = rv
    idx_ref[...] = ri


def _topk2_body(x_ref, vals_ref, idx_ref):
    C = 8192
    rv = ri = None
    for c in range(32768 // C):
        x = x_ref[pl.ds(c * C, C), :]
        iota = jax.lax.broadcasted_iota(jnp.int32, x.shape, 0) + c * C
        lv, li = _local_topk(x, iota, 3, 0, True)
        if rv is None:
            rv, ri = lv, li
        else:
            rv, ri = _lex_merge(
                jnp.concatenate([rv, lv], axis=0),
                jnp.concatenate([ri, li], axis=0),
                3, 0, True)
    vals_ref[...] = rv
    idx_ref[...] = ri


_N = 8192
_NSEG = 16
_SEG = _N // _NSEG
_CHUNKS = _SEG // 16
_NROWS = 4096
_NC = 2
_NT = 16
_G = 2
_SGROWS = _NT * _G
_NSGS = _NROWS // _NC // _SGROWS
_ROWS_T = _NSGS * _G


def _sc_topk1_body(x_hbm, outv_hbm, outi_hbm,
                   spmem, buf, outv, outi, fill0, fill1, pull0, pull1):
    c = lax.axis_index("c")
    t = lax.axis_index("s")
    crow0 = c * (_NROWS // _NC)
    iota = lax.iota(jnp.int32, 16)
    inf16 = jnp.full((16,), jnp.inf, jnp.float32)
    big16 = jnp.full((16,), _IBIG, jnp.int32)
    fillsems = (fill0, fill1)
    pullsems = (pull0, pull1)

    def fill(sg, slot):
        src = x_hbm.at[pl.ds(crow0 + _SGROWS * sg, _SGROWS)]
        pltpu.async_copy(src, spmem.at[slot], fillsems[slot])

    def fill_wait(slot):
        pltpu.make_async_copy(
            x_hbm.at[pl.ds(crow0, _SGROWS)], spmem.at[slot],
            fillsems[slot]).wait()

    def pull(slot):
        pltpu.async_copy(
            spmem.at[slot, pl.ds(t * _G, _G)], buf.at[slot], pullsems[slot])

    def pull_wait(slot):
        pltpu.make_async_copy(
            x_hbm.at[pl.ds(crow0, _G)], buf.at[slot], pullsems[slot]).wait()

    def do_row(rr, slot, r):
        def segstep(j, accs):
            base = j * 16
            return tuple(
                jnp.minimum(accs[s], buf[slot, r, pl.ds(s * _SEG + base, 16)])
                for s in range(_NSEG))

        accs = list(lax.fori_loop(0, _CHUNKS, segstep, (inf16,) * _NSEG))

        codes = [iota + s * 16 for s in range(_NSEG)]
        rank_v, rank_s, rank_l = [], [], []
        for p in range(5):
            tv = list(accs)
            tc = list(codes)
            while len(tv) > 1:
                nv, nc = [], []
                for q in range(0, len(tv), 2):
                    cc = tv[q + 1] < tv[q]
                    nv.append(jnp.where(cc, tv[q + 1], tv[q]))
                    nc.append(jnp.where(cc, tc[q + 1], tc[q]))
                tv, tc = nv, nc
            m = jnp.min(tv[0])
            if p == 4:
                rank_v.append(m)
                break
            code = jnp.min(jnp.where(tv[0] == m, tc[0], _IBIG))
            lane = lax.bitwise_and(code, 15)
            sid = lax.shift_right_logical(code, 4)
            rank_v.append(m)
            rank_s.append(sid)
            rank_l.append(lane)
            accs = [
                jnp.where((iota == lane) & (sid == s), jnp.inf, accs[s])
                for s in range(_NSEG)]

        slotvec = jnp.full((16,), slot, jnp.int32)
        rvec = jnp.full((16,), r, jnp.int32)
        cand_v, cand_i = [], []
        for tt in range(4):
            base = rank_s[tt] * _SEG + rank_l[tt]
            for h in range(2):
                idx = base + iota * 16 + h * 256
                cand_v.append(plsc.load_gather(buf, [slotvec, rvec, idx]))
                cand_i.append(idx)

        vals4, idx4 = [], []
        cv = list(cand_v)
        for p in range(4):
            tv = list(cv)
            ti = list(cand_i)
            while len(tv) > 1:
                nv, ni = [], []
                for q in range(0, len(tv), 2):
                    cc = (tv[q + 1] < tv[q]) | (
                        (tv[q + 1] == tv[q]) & (ti[q + 1] < ti[q]))
                    nv.append(jnp.where(cc, tv[q + 1], tv[q]))
                    ni.append(jnp.where(cc, ti[q + 1], ti[q]))
                tv, ti = nv, ni
            m = jnp.min(tv[0])
            mi = jnp.min(jnp.where(tv[0] == m, ti[0], _IBIG))
            vals4.append(m)
            idx4.append(mi)
            if p < 3:
                cv = [jnp.where(cand_i[q] == mi, jnp.inf, cv[q])
                      for q in range(8)]

        exact = vals4[3] < rank_v[4]

        def fallback():
            vs, ids = [], []
            for p in range(4):
                def step(j, carry, _prior=tuple(ids)):
                    fbv, fbi = carry
                    v = buf[slot, r, pl.ds(j * 16, 16)]
                    gi = j * 16 + iota
                    for e in _prior:
                        v = jnp.where(gi == e, jnp.inf, v)
                    cc = (v < fbv) | ((v == fbv) & (gi < fbi))
                    return (jnp.where(cc, v, fbv), jnp.where(cc, gi, fbi))

                fbv, fbi = lax.fori_loop(0, _N // 16, step, (inf16, big16))
                m = jnp.min(fbv)
                mi = jnp.min(jnp.where(fbv == m, fbi, _IBIG))
                vs.append(m)
                ids.append(mi)
            return tuple(vs), tuple(ids)

        vals4, idx4 = lax.cond(
            exact, lambda: (tuple(vals4), tuple(idx4)), fallback)

        resv = jnp.zeros((16,), jnp.float32)
        resi = jnp.zeros((16,), jnp.int32)
        for p in range(4):
            resv = jnp.where(iota == p, vals4[p], resv)
            resi = jnp.where(iota == p, idx4[p], resi)
        outv[rr] = resv
        outi[rr] = resi

    @pl.when(t == 0)
    def _():
        fill(jnp.int32(0), 0)

    def super_iter(i, carry):
        for slot in range(2):
            sg = 2 * i + slot
            prev = 1 - slot

            @pl.when(t == 0)
            def _():
                fill_wait(slot)

            plsc.subcore_barrier()
            pull(slot)

            @pl.when(sg > 0)
            def _():
                pull_wait(prev)
                for r in range(_G):
                    do_row((sg - 1) * _G + r, prev, r)

            plsc.subcore_barrier()

            @pl.when(t == 0)
            def _():
                fill(jnp.minimum(sg + 1, _NSGS - 1), prev)
        return carry

    lax.fori_loop(0, _NSGS // 2, super_iter, 0)

    pull_wait(1)
    for r in range(_G):
        do_row((_NSGS - 1) * _G + r, 1, r)

    @pl.when(t == 0)
    def _():
        fill_wait(0)

    wid2 = c * _NT + t
    pltpu.sync_copy(outv, outv_hbm.at[pl.ds(wid2 * _ROWS_T, _ROWS_T)])
    pltpu.sync_copy(outi, outi_hbm.at[pl.ds(wid2 * _ROWS_T, _ROWS_T)])


@jax.jit
def _sc_topk1(x1):
    k = functools.partial(
        pl.kernel,
        out_type=[
            jax.ShapeDtypeStruct((_NROWS, 16), jnp.float32),
            jax.ShapeDtypeStruct((_NROWS, 16), jnp.int32),
        ],
        mesh=plsc.VectorSubcoreMesh(core_axis_name="c", subcore_axis_name="s"),
        compiler_params=pltpu.CompilerParams(needs_layout_passes=False),
        scratch_types=[
            pltpu.VMEM_SHARED((2, _SGROWS, _N), jnp.float32),
            pltpu.VMEM((2, _G, _N), jnp.float32),
            pltpu.VMEM((_ROWS_T, 16), jnp.float32),
            pltpu.VMEM((_ROWS_T, 16), jnp.int32),
            pltpu.SemaphoreType.DMA,
            pltpu.SemaphoreType.DMA,
            pltpu.SemaphoreType.DMA,
            pltpu.SemaphoreType.DMA,
        ],
    )(_sc_topk1_body)
    vp, ip = k(x1)
    vp = vp.reshape(_NC, _NT, _NSGS, _G, 16).transpose(0, 2, 1, 3, 4)
    ip = ip.reshape(_NC, _NT, _NSGS, _G, 16).transpose(0, 2, 1, 3, 4)
    return vp.reshape(_NROWS, 16), ip.reshape(_NROWS, 16)


def kernel(v_0, v_1, v_2):
    v4, v5 = pl.pallas_call(
        _topk0_body,
        out_shape=(
            jax.ShapeDtypeStruct((64, 2), jnp.float32),
            jax.ShapeDtypeStruct((64, 2), jnp.int32),
        ),
    )(v_0)

    x1 = v_1.reshape(_NROWS, _N)
    v7p, v8p = _sc_topk1(x1)
    v7 = v7p[:, :4].reshape(8, 16, 32, 4)
    v8 = v8p[:, :4].reshape(8, 16, 32, 4)

    v10, v11 = pl.pallas_call(
        _topk2_body,
        out_shape=(
            jax.ShapeDtypeStruct((3, 64), jnp.float32),
            jax.ShapeDtypeStruct((3, 64), jnp.int32),
        ),
    )(v_2)

    return (v4, v5, v7, v8, v10, v11)


# speedup vs baseline: 1.6088x; 1.6088x over previous
import functools

import jax
import jax.numpy as jnp
from jax import lax
from jax.experimental import pallas as pl
from jax.experimental.pallas import tpu as pltpu
from jax.experimental.pallas import tpu_sc as plsc

_IBIG = 2**30


def _local_topk(x, iota, k, axis, largest):
    vals, idxs = [], []
    for j in range(k):
        if largest:
            m = jnp.max(x, axis=axis, keepdims=True)
        else:
            m = jnp.min(x, axis=axis, keepdims=True)
        i = jnp.min(jnp.where(x == m, iota, _IBIG), axis=axis, keepdims=True)
        vals.append(m)
        idxs.append(i)
        if j < k - 1:
            fill = -jnp.inf if largest else jnp.inf
            x = jnp.where(iota == i, fill, x)
    return jnp.concatenate(vals, axis=axis), jnp.concatenate(idxs, axis=axis)


def _lex_merge(vals, idxs, k, axis, largest):
    out_v, out_i = [], []
    for j in range(k):
        if largest:
            m = jnp.max(vals, axis=axis, keepdims=True)
        else:
            m = jnp.min(vals, axis=axis, keepdims=True)
        i = jnp.min(jnp.where(vals == m, idxs, _IBIG), axis=axis, keepdims=True)
        out_v.append(m)
        out_i.append(i)
        if j < k - 1:
            fill = -jnp.inf if largest else jnp.inf
            vals = jnp.where((vals == m) & (idxs == i), fill, vals)
    return jnp.concatenate(out_v, axis=axis), jnp.concatenate(out_i, axis=axis)


def _topk0_body(x_ref, vals_ref, idx_ref):
    C = 8192
    rv = ri = None
    for c in range(32768 // C):
        x = x_ref[:, pl.ds(c * C, C)]
        iota = jax.lax.broadcasted_iota(jnp.int32, x.shape, 1) + c * C
        lv, li = _local_topk(x, iota, 2, 1, True)
        if rv is None:
            rv, ri = lv, li
        else:
            rv, ri = _lex_merge(
                jnp.concatenate([rv, lv], axis=1),
                jnp.concatenate([ri, li], axis=1),
                2, 1, True)
    vals_ref[...] = rv
    idx_ref[...] = ri


def _topk2_body(x_ref, vals_ref, idx_ref):
    C = 8192
    rv = ri = None
    for c in range(32768 // C):
        x = x_ref[pl.ds(c * C, C), :]
        iota = jax.lax.broadcasted_iota(jnp.int32, x.shape, 0) + c * C
        lv, li = _local_topk(x, iota, 3, 0, True)
        if rv is None:
            rv, ri = lv, li
        else:
            rv, ri = _lex_merge(
                jnp.concatenate([rv, lv], axis=0),
                jnp.concatenate([ri, li], axis=0),
                3, 0, True)
    vals_ref[...] = rv
    idx_ref[...] = ri


_N = 8192
_NSEG = 16
_SEG = _N // _NSEG
_CHUNKS = _SEG // 16
_NROWS = 4096
_NC = 2
_NT = 16
_G = 2
_SGROWS = _NT * _G
_NSGS = _NROWS // _NC // _SGROWS
_ROWS_T = _NSGS * _G


def _sc_topk1_body(x_hbm, outv_hbm, outi_hbm,
                   spmem, buf, outv, outi, fill0, fill1, pull0, pull1):
    c = lax.axis_index("c")
    t = lax.axis_index("s")
    crow0 = c * (_NROWS // _NC)
    iota = lax.iota(jnp.int32, 16)
    inf16 = jnp.full((16,), jnp.inf, jnp.float32)
    big16 = jnp.full((16,), _IBIG, jnp.int32)
    fillsems = (fill0, fill1)
    pullsems = (pull0, pull1)

    def fill(sg, slot):
        src = x_hbm.at[pl.ds(crow0 + _SGROWS * sg + _G * t, _G)]
        pltpu.async_copy(src, spmem.at[slot, pl.ds(_G * t, _G)],
                         fillsems[slot])

    def fill_wait(slot):
        pltpu.make_async_copy(
            x_hbm.at[pl.ds(crow0, _G)], spmem.at[slot, pl.ds(_G * t, _G)],
            fillsems[slot]).wait()

    def pull(slot):
        pltpu.async_copy(
            spmem.at[slot, pl.ds(t * _G, _G)], buf.at[slot], pullsems[slot])

    def pull_wait(slot):
        pltpu.make_async_copy(
            x_hbm.at[pl.ds(crow0, _G)], buf.at[slot], pullsems[slot]).wait()

    def do_row(rr, slot, r):
        def segstep(j, accs):
            base = j * 16
            return tuple(
                jnp.minimum(accs[s], buf[slot, r, pl.ds(s * _SEG + base, 16)])
                for s in range(_NSEG))

        accs = list(lax.fori_loop(0, _CHUNKS, segstep, (inf16,) * _NSEG))

        codes = [iota + s * 16 for s in range(_NSEG)]
        rank_v, rank_s, rank_l = [], [], []
        for p in range(5):
            tv = list(accs)
            tc = list(codes)
            while len(tv) > 1:
                nv, nc = [], []
                for q in range(0, len(tv), 2):
                    cc = tv[q + 1] < tv[q]
                    nv.append(jnp.where(cc, tv[q + 1], tv[q]))
                    nc.append(jnp.where(cc, tc[q + 1], tc[q]))
                tv, tc = nv, nc
            m = jnp.min(tv[0])
            if p == 4:
                rank_v.append(m)
                break
            code = jnp.min(jnp.where(tv[0] == m, tc[0], _IBIG))
            lane = lax.bitwise_and(code, 15)
            sid = lax.shift_right_logical(code, 4)
            rank_v.append(m)
            rank_s.append(sid)
            rank_l.append(lane)
            accs = [
                jnp.where((iota == lane) & (sid == s), jnp.inf, accs[s])
                for s in range(_NSEG)]

        slotvec = jnp.full((16,), slot, jnp.int32)
        rvec = jnp.full((16,), r, jnp.int32)
        cand_v, cand_i = [], []
        for tt in range(4):
            base = rank_s[tt] * _SEG + rank_l[tt]
            for h in range(2):
                idx = base + iota * 16 + h * 256
                cand_v.append(plsc.load_gather(buf, [slotvec, rvec, idx]))
                cand_i.append(idx)

        vals4, idx4 = [], []
        cv = list(cand_v)
        for p in range(4):
            tv = list(cv)
            ti = list(cand_i)
            while len(tv) > 1:
                nv, ni = [], []
                for q in range(0, len(tv), 2):
                    cc = (tv[q + 1] < tv[q]) | (
                        (tv[q + 1] == tv[q]) & (ti[q + 1] < ti[q]))
                    nv.append(jnp.where(cc, tv[q + 1], tv[q]))
                    ni.append(jnp.where(cc, ti[q + 1], ti[q]))
                tv, ti = nv, ni
            m = jnp.min(tv[0])
            mi = jnp.min(jnp.where(tv[0] == m, ti[0], _IBIG))
            vals4.append(m)
            idx4.append(mi)
            if p < 3:
                cv = [jnp.where(cand_i[q] == mi, jnp.inf, cv[q])
                      for q in range(8)]

        exact = vals4[3] < rank_v[4]

        def fallback():
            vs, ids = [], []
            for p in range(4):
                def step(j, carry, _prior=tuple(ids)):
                    fbv, fbi = carry
                    v = buf[slot, r, pl.ds(j * 16, 16)]
                    gi = j * 16 + iota
                    for e in _prior:
                        v = jnp.where(gi == e, jnp.inf, v)
                    cc = (v < fbv) | ((v == fbv) & (gi < fbi))
                    return (jnp.where(cc, v, fbv), jnp.where(cc, gi, fbi))

                fbv, fbi = lax.fori_loop(0, _N // 16, step, (inf16, big16))
                m = jnp.min(fbv)
                mi = jnp.min(jnp.where(fbv == m, fbi, _IBIG))
                vs.append(m)
                ids.append(mi)
            return tuple(vs), tuple(ids)

        vals4, idx4 = lax.cond(
            exact, lambda: (tuple(vals4), tuple(idx4)), fallback)

        resv = jnp.zeros((16,), jnp.float32)
        resi = jnp.zeros((16,), jnp.int32)
        for p in range(4):
            resv = jnp.where(iota == p, vals4[p], resv)
            resi = jnp.where(iota == p, idx4[p], resi)
        outv[rr] = resv
        outi[rr] = resi

    fill(jnp.int32(0), 0)
    fill(jnp.int32(1), 1)

    def super_iter(i, carry):
        for slot in range(2):
            sg = 2 * i + slot
            prev = 1 - slot

            fill_wait(slot)
            pull(slot)

            @pl.when(sg > 0)
            def _():
                pull_wait(prev)
                fill(jnp.minimum(sg + 1, _NSGS - 1), prev)
                for r in range(_G):
                    do_row((sg - 1) * _G + r, prev, r)
        return carry

    lax.fori_loop(0, _NSGS // 2, super_iter, 0)

    pull_wait(1)
    for r in range(_G):
        do_row((_NSGS - 1) * _G + r, 1, r)
    fill_wait(0)

    wid2 = c * _NT + t
    pltpu.sync_copy(outv, outv_hbm.at[pl.ds(wid2 * _ROWS_T, _ROWS_T)])
    pltpu.sync_copy(outi, outi_hbm.at[pl.ds(wid2 * _ROWS_T, _ROWS_T)])


@jax.jit
def _sc_topk1(x1):
    k = functools.partial(
        pl.kernel,
        out_type=[
            jax.ShapeDtypeStruct((_NROWS, 16), jnp.float32),
            jax.ShapeDtypeStruct((_NROWS, 16), jnp.int32),
        ],
        mesh=plsc.VectorSubcoreMesh(core_axis_name="c", subcore_axis_name="s"),
        compiler_params=pltpu.CompilerParams(needs_layout_passes=False),
        scratch_types=[
            pltpu.VMEM_SHARED((2, _SGROWS, _N), jnp.float32),
            pltpu.VMEM((2, _G, _N), jnp.float32),
            pltpu.VMEM((_ROWS_T, 16), jnp.float32),
            pltpu.VMEM((_ROWS_T, 16), jnp.int32),
            pltpu.SemaphoreType.DMA,
            pltpu.SemaphoreType.DMA,
            pltpu.SemaphoreType.DMA,
            pltpu.SemaphoreType.DMA,
        ],
    )(_sc_topk1_body)
    vp, ip = k(x1)
    vp = vp.reshape(_NC, _NT, _NSGS, _G, 16).transpose(0, 2, 1, 3, 4)
    ip = ip.reshape(_NC, _NT, _NSGS, _G, 16).transpose(0, 2, 1, 3, 4)
    return vp.reshape(_NROWS, 16), ip.reshape(_NROWS, 16)


def kernel(v_0, v_1, v_2):
    v4, v5 = pl.pallas_call(
        _topk0_body,
        out_shape=(
            jax.ShapeDtypeStruct((64, 2), jnp.float32),
            jax.ShapeDtypeStruct((64, 2), jnp.int32),
        ),
    )(v_0)

    x1 = v_1.reshape(_NROWS, _N)
    v7p, v8p = _sc_topk1(x1)
    v7 = v7p[:, :4].reshape(8, 16, 32, 4)
    v8 = v8p[:, :4].reshape(8, 16, 32, 4)

    v10, v11 = pl.pallas_call(
        _topk2_body,
        out_shape=(
            jax.ShapeDtypeStruct((3, 64), jnp.float32),
            jax.ShapeDtypeStruct((3, 64), jnp.int32),
        ),
    )(v_2)

    return (v4, v5, v7, v8, v10, v11)
